# baseline (speedup 1.0000x reference)
import functools

import jax
import jax.numpy as jnp
from jax import lax
from jax.experimental import pallas as pl
from jax.experimental.pallas import tpu as pltpu
from jax.experimental.pallas import tpu_sc as plsc

N_ATOMS = 4096
D_MODEL = 1024
HIDDEN = 1024
B_SYS = 128
N_EXPERTS = 4

TILE = 128
PN = N_ATOMS + N_EXPERTS * TILE
G = PN // TILE
L = 16
TE_N = ((PN // TILE) + L - 1) // L * L

D2 = D_MODEL // 2
NW = 32
BPW = PN // NW
CH = 48
NCH = BPW // CH

_MESH = plsc.VectorSubcoreMesh(core_axis_name="c", subcore_axis_name="s")


def _pack_kernel(x_ref, out_ref):
    a = jax.lax.bitcast_convert_type(x_ref[:, :D2], jnp.int32)
    b = jax.lax.bitcast_convert_type(x_ref[:, D2:], jnp.int32)
    lo = jax.lax.shift_right_logical(a + 0x8000, 16)
    hi = (b + 0x8000) & jnp.int32(-65536)
    out_ref[:, :] = lo | hi


def _pack(x):
    return pl.pallas_call(
        _pack_kernel,
        grid=(8,),
        in_specs=[pl.BlockSpec((N_ATOMS // 8, D_MODEL), lambda i: (i, 0))],
        out_specs=pl.BlockSpec((N_ATOMS // 8, D2), lambda i: (i, 0)),
        out_shape=jax.ShapeDtypeStruct((N_ATOMS, D2), jnp.int32),
    )(x)


def _route_body(batch_hbm, ds_hbm, src_hbm, bp_hbm, te_hbm,
                batch_v, ds_v, src_v, bp_v, te_v):
    cid = lax.axis_index("c")
    sid = lax.axis_index("s")

    @pl.when((cid == 0) & (sid == 0))
    def _():
        pltpu.sync_copy(batch_hbm, batch_v)
        pltpu.sync_copy(ds_hbm, ds_v)

        zero16 = jnp.zeros((L,), jnp.int32)
        neg16 = jnp.full((L,), -1, jnp.int32)

        def init_body(i, carry):
            src_v[pl.ds(i * L, L)] = zero16
            bp_v[pl.ds(i * L, L)] = neg16
            return carry

        lax.fori_loop(0, PN // L, init_body, 0)

        def count_body(i, carry):
            c0, c1, c2, c3 = carry
            b = batch_v[pl.ds(i * L, L)]
            e = plsc.load_gather(ds_v, [b])
            c0 = c0 + jnp.sum((e == 0).astype(jnp.int32))
            c1 = c1 + jnp.sum((e == 1).astype(jnp.int32))
            c2 = c2 + jnp.sum((e == 2).astype(jnp.int32))
            c3 = c3 + jnp.sum((e == 3).astype(jnp.int32))
            return c0, c1, c2, c3

        z = jnp.int32(0)
        c0, c1, c2, c3 = lax.fori_loop(0, N_ATOMS // L, count_body,
                                       (z, z, z, z))

        def _pad(c):
            return ((c + TILE - 1) // TILE) * TILE

        off1 = _pad(c0)
        off2 = off1 + _pad(c1)
        off3 = off2 + _pad(c2)

        gv = jnp.arange(L, dtype=jnp.int32) * TILE
        for half in range(TE_N // L):
            g = gv + half * L * TILE
            te = ((g >= off1).astype(jnp.int32)
                  + (g >= off2).astype(jnp.int32)
                  + (g >= off3).astype(jnp.int32))
            te_v[pl.ds(half * L, L)] = te

        lane = jnp.arange(L, dtype=jnp.int32)

        def place_body(i, carry):
            r0, r1, r2, r3 = carry
            b = batch_v[pl.ds(i * L, L)]
            e = plsc.load_gather(ds_v, [b])
            dst = jnp.zeros((L,), jnp.int32)
            runs = [r0, r1, r2, r3]
            new_runs = []
            for ex in range(N_EXPERTS):
                m = (e == ex)
                mi = m.astype(jnp.int32)
                pref = plsc.cumsum(mi) - 1
                dst = jnp.where(m, runs[ex] + pref, dst)
                new_runs.append(runs[ex] + jnp.sum(mi))
            atomid = i * L + lane
            plsc.store_scatter(src_v, [dst], atomid)
            plsc.store_scatter(bp_v, [dst], b)
            return tuple(new_runs)

        lax.fori_loop(0, N_ATOMS // L, place_body,
                      (jnp.int32(0), off1, off2, off3))

        pltpu.sync_copy(src_v, src_hbm)
        pltpu.sync_copy(bp_v, bp_hbm)
        pltpu.sync_copy(te_v, te_hbm)


_route = functools.partial(
    pl.kernel,
    out_type=[jax.ShapeDtypeStruct((PN,), jnp.int32),
              jax.ShapeDtypeStruct((PN,), jnp.int32),
              jax.ShapeDtypeStruct((TE_N,), jnp.int32)],
    mesh=_MESH,
    scratch_types=[pltpu.VMEM((N_ATOMS,), jnp.int32),
                   pltpu.VMEM((B_SYS,), jnp.int32),
                   pltpu.VMEM((PN,), jnp.int32),
                   pltpu.VMEM((PN,), jnp.int32),
                   pltpu.VMEM((TE_N,), jnp.int32)],
    compiler_params=pltpu.CompilerParams(needs_layout_passes=False),
)(_route_body)


SPLIT_TILES = (8, 20, 8)
SPLIT_ROWS = tuple(t * TILE for t in SPLIT_TILES)
SPLIT_ROW_BASE = tuple(sum(SPLIT_ROWS[:i]) for i in range(len(SPLIT_TILES)))
SPLIT_TILE_BASE = tuple(sum(SPLIT_TILES[:i]) for i in range(len(SPLIT_TILES)))


def _make_gather_body(row_base, bpw):
    def _gather_body(x_hbm, src_hbm, xs_hbm, idx_v, rows_v, sem):
        cid = lax.axis_index("c")
        sid = lax.axis_index("s")
        wid = sid * 2 + cid
        base = wid * bpw
        pltpu.sync_copy(src_hbm.at[pl.ds(row_base + base, bpw)], idx_v)
        pltpu.async_copy(x_hbm.at[idx_v], rows_v, sem).wait()
        pltpu.sync_copy(rows_v, xs_hbm.at[pl.ds(base, bpw), :])
    return _gather_body


def _make_gather(row_base, hpn):
    bpw = hpn // NW
    return functools.partial(
        pl.kernel,
        out_type=jax.ShapeDtypeStruct((hpn, D2), jnp.int32),
        mesh=_MESH,
        scratch_types=[pltpu.VMEM((bpw,), jnp.int32),
                       pltpu.VMEM((bpw, D2), jnp.int32),
                       pltpu.SemaphoreType.DMA],
    )(_make_gather_body(row_base, bpw))


_gather_h = tuple(_make_gather(SPLIT_ROW_BASE[i], SPLIT_ROWS[i])
                  for i in range(len(SPLIT_TILES)))


def _make_mlp_tile_kernel(tile_base):
  def _mlp_tile_kernel(tile_eid_ref, xs_ref, w1_ref, b1_ref, w2_ref, b2_ref,
                       bpad_ref, ds_ref, out_ref):
    i = pl.program_id(0) + tile_base

    @pl.when(pl.program_id(0) == 0)
    def _init():
        out_ref[:, :] = jnp.zeros_like(out_ref)

    w = xs_ref[:, :]
    xa = jax.lax.bitcast_convert_type(
        jax.lax.shift_left(w, 16), jnp.float32).astype(jnp.bfloat16)
    xb = jax.lax.bitcast_convert_type(
        w & jnp.int32(-65536), jnp.float32).astype(jnp.bfloat16)
    w1 = w1_ref[0].astype(jnp.bfloat16)
    h = (jnp.dot(xa, w1[:D2, :], preferred_element_type=jnp.float32)
         + jnp.dot(xb, w1[D2:, :], preferred_element_type=jnp.float32))
    h = h + b1_ref[0]
    h = jax.nn.gelu(h)
    w2 = w2_ref[0]
    e = jnp.sum(h * w2, axis=1, keepdims=True)
    eid = tile_eid_ref[i]
    e = e + b2_ref[eid, 0]

    bp = bpad_ref[:, :]
    sys_iota = jax.lax.broadcasted_iota(jnp.int32, (TILE, B_SYS), 1)
    A = (bp == sys_iota).astype(jnp.float32)
    contrib = jax.lax.dot_general(e, A, (((0,), (0,)), ((), ())),
                                  preferred_element_type=jnp.float32)
    ds = ds_ref[:, :]
    d_iota = jax.lax.broadcasted_iota(jnp.int32, (N_EXPERTS, B_SYS), 0)
    dmask = (ds == d_iota).astype(jnp.float32)
    out_ref[:, :] += dmask * contrib
  return _mlp_tile_kernel


def _routed_mlp_half(half, tile_eid, xs_h, W1, b1, W2r, b2, batch_pad, ds_row):
    tile_base = SPLIT_TILE_BASE[half]
    grid_spec = pltpu.PrefetchScalarGridSpec(
        num_scalar_prefetch=1,
        grid=(SPLIT_TILES[half],),
        in_specs=[
            pl.BlockSpec((TILE, D2), lambda i, s: (i, 0)),
            pl.BlockSpec((1, D_MODEL, HIDDEN),
                         lambda i, s: (s[i + tile_base], 0, 0)),
            pl.BlockSpec((1, 1, HIDDEN),
                         lambda i, s: (s[i + tile_base], 0, 0)),
            pl.BlockSpec((1, 1, HIDDEN),
                         lambda i, s: (s[i + tile_base], 0, 0)),
            pl.BlockSpec(memory_space=pltpu.SMEM),
            pl.BlockSpec((TILE, 1), lambda i, s: (i + tile_base * 1, 0)),
            pl.BlockSpec((1, B_SYS), lambda i, s: (0, 0)),
        ],
        out_specs=pl.BlockSpec((N_EXPERTS, B_SYS), lambda i, s: (0, 0)),
    )
    return pl.pallas_call(
        _make_mlp_tile_kernel(tile_base),
        grid_spec=grid_spec,
        out_shape=jax.ShapeDtypeStruct((N_EXPERTS, B_SYS), jnp.float32),
    )(tile_eid, xs_h, W1, b1, W2r, b2, batch_pad, ds_row)


def kernel(x, batch, dataset_ids, W1, b1, W2, b2):
    batch32 = batch.astype(jnp.int32)
    ds32 = dataset_ids.astype(jnp.int32)

    src, batch_pad, tile_eid = _route(batch32, ds32)
    bp2 = batch_pad.reshape(PN, 1)
    ds_row = ds32.reshape(1, B_SYS)
    b1r = b1.reshape(N_EXPERTS, 1, HIDDEN)
    W2r = jnp.transpose(W2, (0, 2, 1))

    xp = _pack(x)
    out = None
    for h in range(len(SPLIT_TILES)):
        xs_h = _gather_h[h](xp, src)
        part = _routed_mlp_half(h, tile_eid, xs_h, W1, b1r, W2r, b2,
                                bp2, ds_row)
        out = part if out is None else out + part
    return out


# speedup vs baseline: 1.0272x; 1.0272x over previous
import functools

import jax
import jax.numpy as jnp
from jax import lax
from jax.experimental import pallas as pl
from jax.experimental.pallas import tpu as pltpu
from jax.experimental.pallas import tpu_sc as plsc

N_ATOMS = 4096
D_MODEL = 1024
HIDDEN = 1024
B_SYS = 128
N_EXPERTS = 4

TILE = 128
PN = N_ATOMS + N_EXPERTS * TILE
G = PN // TILE
L = 16
TE_N = ((PN // TILE) + L - 1) // L * L

D2 = D_MODEL // 2
NW = 32
BPW = PN // NW
CH = 48
NCH = BPW // CH

_MESH = plsc.VectorSubcoreMesh(core_axis_name="c", subcore_axis_name="s")


def _pack_kernel(x_ref, out_ref):
    a = jax.lax.bitcast_convert_type(x_ref[:, :D2], jnp.int32)
    b = jax.lax.bitcast_convert_type(x_ref[:, D2:], jnp.int32)
    lo = jax.lax.shift_right_logical(a + 0x8000, 16)
    hi = (b + 0x8000) & jnp.int32(-65536)
    out_ref[:, :] = lo | hi


def _pack(x):
    return pl.pallas_call(
        _pack_kernel,
        grid=(8,),
        in_specs=[pl.BlockSpec((N_ATOMS // 8, D_MODEL), lambda i: (i, 0))],
        out_specs=pl.BlockSpec((N_ATOMS // 8, D2), lambda i: (i, 0)),
        out_shape=jax.ShapeDtypeStruct((N_ATOMS, D2), jnp.int32),
    )(x)


def _route_body(batch_hbm, ds_hbm, src_hbm, bp_hbm, te_hbm,
                batch_v, ds_v, src_v, bp_v, te_v):
    cid = lax.axis_index("c")
    sid = lax.axis_index("s")

    @pl.when((cid == 0) & (sid == 0))
    def _():
        pltpu.sync_copy(batch_hbm, batch_v)
        pltpu.sync_copy(ds_hbm, ds_v)

        zero16 = jnp.zeros((L,), jnp.int32)
        neg16 = jnp.full((L,), -1, jnp.int32)

        def init_body(i, carry):
            src_v[pl.ds(i * L, L)] = zero16
            bp_v[pl.ds(i * L, L)] = neg16
            return carry

        lax.fori_loop(0, PN // L, init_body, 0)

        def count_body(i, carry):
            c0, c1, c2, c3 = carry
            b = batch_v[pl.ds(i * L, L)]
            e = plsc.load_gather(ds_v, [b])
            c0 = c0 + jnp.sum((e == 0).astype(jnp.int32))
            c1 = c1 + jnp.sum((e == 1).astype(jnp.int32))
            c2 = c2 + jnp.sum((e == 2).astype(jnp.int32))
            c3 = c3 + jnp.sum((e == 3).astype(jnp.int32))
            return c0, c1, c2, c3

        z = jnp.int32(0)
        c0, c1, c2, c3 = lax.fori_loop(0, N_ATOMS // L, count_body,
                                       (z, z, z, z))

        def _pad(c):
            return ((c + TILE - 1) // TILE) * TILE

        off1 = _pad(c0)
        off2 = off1 + _pad(c1)
        off3 = off2 + _pad(c2)

        gv = jnp.arange(L, dtype=jnp.int32) * TILE
        for half in range(TE_N // L):
            g = gv + half * L * TILE
            te = ((g >= off1).astype(jnp.int32)
                  + (g >= off2).astype(jnp.int32)
                  + (g >= off3).astype(jnp.int32))
            te_v[pl.ds(half * L, L)] = te

        lane = jnp.arange(L, dtype=jnp.int32)

        def place_body(i, carry):
            r0, r1, r2, r3 = carry
            b = batch_v[pl.ds(i * L, L)]
            e = plsc.load_gather(ds_v, [b])
            dst = jnp.zeros((L,), jnp.int32)
            runs = [r0, r1, r2, r3]
            new_runs = []
            for ex in range(N_EXPERTS):
                m = (e == ex)
                mi = m.astype(jnp.int32)
                pref = plsc.cumsum(mi) - 1
                dst = jnp.where(m, runs[ex] + pref, dst)
                new_runs.append(runs[ex] + jnp.sum(mi))
            atomid = i * L + lane
            plsc.store_scatter(src_v, [dst], atomid)
            plsc.store_scatter(bp_v, [dst], b)
            return tuple(new_runs)

        lax.fori_loop(0, N_ATOMS // L, place_body,
                      (jnp.int32(0), off1, off2, off3))

        pltpu.sync_copy(src_v, src_hbm)
        pltpu.sync_copy(bp_v, bp_hbm)
        pltpu.sync_copy(te_v, te_hbm)


_route = functools.partial(
    pl.kernel,
    out_type=[jax.ShapeDtypeStruct((PN,), jnp.int32),
              jax.ShapeDtypeStruct((PN,), jnp.int32),
              jax.ShapeDtypeStruct((TE_N,), jnp.int32)],
    mesh=_MESH,
    scratch_types=[pltpu.VMEM((N_ATOMS,), jnp.int32),
                   pltpu.VMEM((B_SYS,), jnp.int32),
                   pltpu.VMEM((PN,), jnp.int32),
                   pltpu.VMEM((PN,), jnp.int32),
                   pltpu.VMEM((TE_N,), jnp.int32)],
    compiler_params=pltpu.CompilerParams(needs_layout_passes=False),
)(_route_body)


NSPLIT = 3
SPLIT_TILES = tuple(G // NSPLIT for _ in range(NSPLIT))
SPLIT_ROWS = tuple(t * TILE for t in SPLIT_TILES)
SPLIT_ROW_BASE = tuple(sum(SPLIT_ROWS[:i]) for i in range(len(SPLIT_TILES)))
SPLIT_TILE_BASE = tuple(sum(SPLIT_TILES[:i]) for i in range(len(SPLIT_TILES)))
HPN = PN // NSPLIT
HBPW = HPN // NW
HNCH = HBPW // CH


def _make_gather_body(row_base):
    def _gather_body(x_hbm, src_hbm, xs_hbm, idx_v, rows_a, rows_b,
                     sem_a, sem_b):
        cid = lax.axis_index("c")
        sid = lax.axis_index("s")
        wid = sid * 2 + cid
        base = wid * HBPW
        pltpu.sync_copy(src_hbm.at[pl.ds(row_base + base, HBPW)], idx_v)

        bufs = (rows_a, rows_b)
        sems = (sem_a, sem_b)
        handles = []
        for j in range(HNCH):
            handles.append(pltpu.async_copy(
                x_hbm.at[idx_v.at[pl.ds(j * CH, CH)]],
                bufs[j % 2], sems[j % 2]))
            if j >= 1:
                handles[j - 1].wait()
                pltpu.sync_copy(bufs[(j - 1) % 2],
                                xs_hbm.at[pl.ds(base + (j - 1) * CH, CH), :])
        handles[HNCH - 1].wait()
        pltpu.sync_copy(bufs[(HNCH - 1) % 2],
                        xs_hbm.at[pl.ds(base + (HNCH - 1) * CH, CH), :])
    return _gather_body


def _make_gather(row_base):
    return functools.partial(
        pl.kernel,
        out_type=jax.ShapeDtypeStruct((HPN, D2), jnp.int32),
        mesh=_MESH,
        scratch_types=[pltpu.VMEM((HBPW,), jnp.int32),
                       pltpu.VMEM((CH, D2), jnp.int32),
                       pltpu.VMEM((CH, D2), jnp.int32),
                       pltpu.SemaphoreType.DMA,
                       pltpu.SemaphoreType.DMA],
    )(_make_gather_body(row_base))


_gather_h = tuple(_make_gather(h * HPN) for h in range(NSPLIT))


def _make_mlp_tile_kernel(tile_base):
  def _mlp_tile_kernel(tile_eid_ref, xs_ref, w1_ref, b1_ref, w2_ref, b2_ref,
                       bpad_ref, ds_ref, out_ref):
    i = pl.program_id(0) + tile_base

    @pl.when(pl.program_id(0) == 0)
    def _init():
        out_ref[:, :] = jnp.zeros_like(out_ref)

    w = xs_ref[:, :]
    xa = jax.lax.bitcast_convert_type(
        jax.lax.shift_left(w, 16), jnp.float32).astype(jnp.bfloat16)
    xb = jax.lax.bitcast_convert_type(
        w & jnp.int32(-65536), jnp.float32).astype(jnp.bfloat16)
    w1 = w1_ref[0].astype(jnp.bfloat16)
    h = (jnp.dot(xa, w1[:D2, :], preferred_element_type=jnp.float32)
         + jnp.dot(xb, w1[D2:, :], preferred_element_type=jnp.float32))
    h = h + b1_ref[0]
    h = jax.nn.gelu(h)
    w2 = w2_ref[0]
    e = jnp.sum(h * w2, axis=1, keepdims=True)
    eid = tile_eid_ref[i]
    e = e + b2_ref[eid, 0]

    bp = bpad_ref[:, :]
    sys_iota = jax.lax.broadcasted_iota(jnp.int32, (TILE, B_SYS), 1)
    A = (bp == sys_iota).astype(jnp.float32)
    contrib = jax.lax.dot_general(e, A, (((0,), (0,)), ((), ())),
                                  preferred_element_type=jnp.float32)
    ds = ds_ref[:, :]
    d_iota = jax.lax.broadcasted_iota(jnp.int32, (N_EXPERTS, B_SYS), 0)
    dmask = (ds == d_iota).astype(jnp.float32)
    out_ref[:, :] += dmask * contrib
  return _mlp_tile_kernel


def _routed_mlp_half(half, tile_eid, xs_h, W1, b1, W2r, b2, batch_pad, ds_row):
    tile_base = SPLIT_TILE_BASE[half]
    grid_spec = pltpu.PrefetchScalarGridSpec(
        num_scalar_prefetch=1,
        grid=(SPLIT_TILES[half],),
        in_specs=[
            pl.BlockSpec((TILE, D2), lambda i, s: (i, 0)),
            pl.BlockSpec((1, D_MODEL, HIDDEN),
                         lambda i, s: (s[i + tile_base], 0, 0)),
            pl.BlockSpec((1, 1, HIDDEN),
                         lambda i, s: (s[i + tile_base], 0, 0)),
            pl.BlockSpec((1, 1, HIDDEN),
                         lambda i, s: (s[i + tile_base], 0, 0)),
            pl.BlockSpec(memory_space=pltpu.SMEM),
            pl.BlockSpec((TILE, 1), lambda i, s: (i + tile_base * 1, 0)),
            pl.BlockSpec((1, B_SYS), lambda i, s: (0, 0)),
        ],
        out_specs=pl.BlockSpec((N_EXPERTS, B_SYS), lambda i, s: (0, 0)),
    )
    return pl.pallas_call(
        _make_mlp_tile_kernel(tile_base),
        grid_spec=grid_spec,
        out_shape=jax.ShapeDtypeStruct((N_EXPERTS, B_SYS), jnp.float32),
    )(tile_eid, xs_h, W1, b1, W2r, b2, batch_pad, ds_row)


def kernel(x, batch, dataset_ids, W1, b1, W2, b2):
    batch32 = batch.astype(jnp.int32)
    ds32 = dataset_ids.astype(jnp.int32)

    src, batch_pad, tile_eid = _route(batch32, ds32)
    bp2 = batch_pad.reshape(PN, 1)
    ds_row = ds32.reshape(1, B_SYS)
    b1r = b1.reshape(N_EXPERTS, 1, HIDDEN)
    W2r = jnp.transpose(W2, (0, 2, 1))

    xp = _pack(x)
    out = None
    for h in range(len(SPLIT_TILES)):
        xs_h = _gather_h[h](xp, src)
        part = _routed_mlp_half(h, tile_eid, xs_h, W1, b1r, W2r, b2,
                                bp2, ds_row)
        out = part if out is None else out + part
    return out
